# baseline (speedup 1.0000x reference)
import functools

import jax
import jax.numpy as jnp
from jax import lax
from jax.experimental import pallas as pl
from jax.experimental.pallas import tpu as pltpu
from jax.experimental.pallas import tpu_sc as plsc

VOCAB = 32000
D_MODEL = 512
BATCH = 64
SEQ = 512

NC = 2
NS = 16
NW = NC * NS
ROWS = BATCH * SEQ
RPW = ROWS // NW
CHUNK = 64
NCHUNK = RPW // CHUNK
PCHUNK = SEQ // CHUNK
LANES = 16
GRP = D_MODEL // LANES


def _positional_encoding():
    pos = jnp.arange(SEQ, dtype=jnp.float32)[:, None]
    i = jnp.arange(D_MODEL, dtype=jnp.float32)[None, :]
    angle = pos / jnp.power(10000.0, 2.0 * i / D_MODEL)
    even = (jnp.arange(D_MODEL) % 2 == 0)[None, :]
    return jnp.where(even, jnp.sin(angle), jnp.cos(angle)).astype(jnp.float32)


_mesh = plsc.VectorSubcoreMesh(core_axis_name="c", subcore_axis_name="s")


@functools.partial(
    pl.kernel,
    mesh=_mesh,
    out_type=jax.ShapeDtypeStruct((ROWS, D_MODEL), jnp.float32),
    scratch_types=[
        pltpu.VMEM((NCHUNK, CHUNK), jnp.int32),
        pltpu.VMEM((CHUNK, D_MODEL), jnp.float32),
        pltpu.VMEM((CHUNK, D_MODEL), jnp.float32),
        pltpu.SemaphoreType.DMA,
    ],
)
def _emb_kernel(x_hbm, table_hbm, pe_hbm, out_hbm, idx_v, buf, pe_v, sem):
    wid = lax.axis_index("s") * NC + lax.axis_index("c")
    base = wid * RPW
    pltpu.sync_copy(x_hbm.at[wid], idx_v)
    for p in range(PCHUNK):
        pltpu.sync_copy(pe_hbm.at[pl.ds(p * CHUNK, CHUNK)], pe_v)
        for seq in range(2):
            k = seq * PCHUNK + p
            pltpu.async_copy(table_hbm.at[idx_v.at[k]], buf, sem).wait()

            def body(i, carry):
                r = i // GRP
                c0 = (i % GRP) * LANES
                buf[r, pl.ds(c0, LANES)] = (
                    buf[r, pl.ds(c0, LANES)] + pe_v[r, pl.ds(c0, LANES)]
                )
                return carry

            lax.fori_loop(0, CHUNK * GRP, body, 0)
            pltpu.sync_copy(buf, out_hbm.at[pl.ds(base + k * CHUNK, CHUNK)])


def kernel(x, table):
    pe = _positional_encoding()
    xf = x.astype(jnp.int32).reshape(NW, NCHUNK, CHUNK)
    out = _emb_kernel(xf, table, pe)
    return out.reshape(BATCH, SEQ, D_MODEL)


# speedup vs baseline: 2.1988x; 2.1988x over previous
import functools

import jax
import jax.numpy as jnp
from jax import lax
from jax.experimental import pallas as pl
from jax.experimental.pallas import tpu as pltpu
from jax.experimental.pallas import tpu_sc as plsc

VOCAB = 32000
D_MODEL = 512
BATCH = 64
SEQ = 512

NC = 2
NS = 16
NW = NC * NS
ROWS = BATCH * SEQ
RPW = ROWS // NW
CHUNK = 32
NCHUNK = RPW // CHUNK
PCHUNK = SEQ // CHUNK
LANES = 16
GRP = D_MODEL // LANES


def _positional_encoding():
    pos = jnp.arange(SEQ, dtype=jnp.float32)[:, None]
    i = jnp.arange(D_MODEL, dtype=jnp.float32)[None, :]
    angle = pos / jnp.power(10000.0, 2.0 * i / D_MODEL)
    even = (jnp.arange(D_MODEL) % 2 == 0)[None, :]
    return jnp.where(even, jnp.sin(angle), jnp.cos(angle)).astype(jnp.float32)


_mesh = plsc.VectorSubcoreMesh(core_axis_name="c", subcore_axis_name="s")


def _chunk_of(i):
    return (i % 2) * PCHUNK + i // 2


@functools.partial(
    pl.kernel,
    mesh=_mesh,
    out_type=jax.ShapeDtypeStruct((ROWS, D_MODEL), jnp.float32),
    scratch_types=[
        pltpu.VMEM((NCHUNK, CHUNK), jnp.int32),
        pltpu.VMEM((CHUNK, D_MODEL), jnp.float32),
        pltpu.VMEM((CHUNK, D_MODEL), jnp.float32),
        pltpu.VMEM((CHUNK, D_MODEL), jnp.float32),
        pltpu.VMEM((CHUNK, D_MODEL), jnp.float32),
        pltpu.VMEM((CHUNK, D_MODEL), jnp.float32),
        pltpu.SemaphoreType.DMA,
        pltpu.SemaphoreType.DMA,
        pltpu.SemaphoreType.DMA,
        pltpu.SemaphoreType.DMA,
    ],
)
def _emb_kernel(x_hbm, table_hbm, pe_hbm, out_hbm, idx_v,
                g0, g1, o0, o1, pe_v, gs0, gs1, os0, os1):
    wid = lax.axis_index("s") * NC + lax.axis_index("c")
    base = wid * RPW
    g = (g0, g1)
    o = (o0, o1)
    gsem = (gs0, gs1)
    osem = (os0, os1)

    pltpu.sync_copy(x_hbm.at[wid], idx_v)

    hg = {}
    ho = {}
    for i in range(2):
        hg[i] = pltpu.async_copy(
            table_hbm.at[idx_v.at[_chunk_of(i)]], g[i], gsem[i])

    for i in range(NCHUNK):
        b = i % 2
        if b == 0:
            pltpu.sync_copy(
                pe_hbm.at[pl.ds((i // 2) * CHUNK, CHUNK)], pe_v)
        hg[i].wait()
        if i >= 2:
            ho[i - 2].wait()

        def addbody(r, carry, _b=b):
            for jg in range(GRP):
                sl = pl.ds(jg * LANES, LANES)
                o[_b][r, sl] = g[_b][r, sl] + pe_v[r, sl]
            return carry

        lax.fori_loop(0, CHUNK, addbody, 0)

        if i + 2 < NCHUNK:
            hg[i + 2] = pltpu.async_copy(
                table_hbm.at[idx_v.at[_chunk_of(i + 2)]], g[b], gsem[b])
        ho[i] = pltpu.async_copy(
            o[b], out_hbm.at[pl.ds(base + _chunk_of(i) * CHUNK, CHUNK)],
            osem[b])

    ho[NCHUNK - 2].wait()
    ho[NCHUNK - 1].wait()


def kernel(x, table):
    pe = _positional_encoding()
    xf = x.astype(jnp.int32).reshape(NW, NCHUNK, CHUNK)
    out = _emb_kernel(xf, table, pe)
    return out.reshape(BATCH, SEQ, D_MODEL)
